# Initial kernel scaffold; baseline (speedup 1.0000x reference)
import jax
import jax.numpy as jnp
from jax.experimental import pallas as pl


def kernel(x, edge_index, weight, W_ih, W_hh, b_ih, b_hh):
    raise NotImplementedError("write your pallas kernel here")


# speedup vs baseline: 6.0843x; 6.0843x over previous
import functools

import jax
import jax.numpy as jnp
from jax import lax
from jax.experimental import pallas as pl
from jax.experimental.pallas import tpu as pltpu
from jax.experimental.pallas import tpu_sc as plsc

NC = 2
NS = 16
NW = NC * NS
C = 128


def _sc_scatter_body(nchunks, trips, n, d, m_hbm, src_hbm, dst_hbm, out_hbm,
                     idx_s, idx_d, rows, agg_sh, sem):
    cid = lax.axis_index("c")
    sid = lax.axis_index("s")
    wid = cid * NS + sid

    def _zrow(i, _):
        def _zlane(j, _):
            rows[i, pl.ds(j * 16, 16)] = jnp.zeros((16,), jnp.float32)
            return 0
        return lax.fori_loop(0, d // 16, _zlane, 0)
    lax.fori_loop(0, C, _zrow, 0)

    rpt = (n // NS) // 8 * 8
    tail = n - rpt * NS
    base = sid * rpt

    def _fill_zeros(b0, cnt):
        nfull, rem = cnt // C, cnt % C
        for k in range(nfull):
            pltpu.sync_copy(rows, agg_sh.at[pl.ds(b0 + k * C, C)])
        if rem:
            pltpu.sync_copy(rows.at[pl.ds(0, rem)],
                            agg_sh.at[pl.ds(b0 + nfull * C, rem)])

    _fill_zeros(base, rpt)
    if tail:
        @pl.when(sid == NS - 1)
        def _():
            _fill_zeros(NS * rpt, tail)
    plsc.subcore_barrier()

    def _chunk(j, _):
        chunk = j * NW + wid

        @pl.when(chunk < nchunks)
        def _():
            off = chunk * C
            pltpu.sync_copy(src_hbm.at[pl.ds(off, C)], idx_s)
            pltpu.sync_copy(dst_hbm.at[pl.ds(off, C)], idx_d)
            pltpu.async_copy(m_hbm.at[idx_s], rows, sem).wait()
            pltpu.sync_copy(rows, agg_sh.at[idx_d], add=True)
        return 0
    lax.fori_loop(0, trips, _chunk, 0)

    plsc.subcore_barrier()
    pltpu.sync_copy(agg_sh.at[pl.ds(base, rpt)],
                    out_hbm.at[pl.ds(cid * n + base, rpt)])
    if tail:
        @pl.when(sid == NS - 1)
        def _():
            pltpu.sync_copy(agg_sh.at[pl.ds(NS * rpt, tail)],
                            out_hbm.at[pl.ds(cid * n + NS * rpt, tail)])


@functools.cache
def _make_sc_scatter(n, e, d):
    assert e % C == 0 and n % NS == 0 and d % 16 == 0
    nchunks = e // C
    trips = (nchunks + NW - 1) // NW
    mesh = plsc.VectorSubcoreMesh(core_axis_name="c", subcore_axis_name="s",
                                  num_cores=NC, num_subcores=NS)
    return pl.kernel(
        functools.partial(_sc_scatter_body, nchunks, trips, n, d),
        out_type=jax.ShapeDtypeStruct((NC * n, d), jnp.float32),
        mesh=mesh,
        scratch_types=[
            pltpu.VMEM((C,), jnp.int32),
            pltpu.VMEM((C,), jnp.int32),
            pltpu.VMEM((C, d), jnp.float32),
            pltpu.VMEM_SHARED((n, d), jnp.float32),
            pltpu.SemaphoreType.DMA,
        ],
    )


def _mm_body(h_ref, w_ref, o_ref):
    o_ref[...] = jnp.dot(h_ref[...], w_ref[...],
                         preferred_element_type=jnp.float32)


def _gru_body(has_next, d, a0_ref, a1_ref, h_ref, wih_ref, whh_ref,
              bih_ref, bhh_ref, wn_ref, ho_ref, mo_ref=None):
    agg = a0_ref[...] + a1_ref[...]
    h = h_ref[...]
    gi = jnp.dot(agg, wih_ref[...], preferred_element_type=jnp.float32) \
        + bih_ref[...]
    gh = jnp.dot(h, whh_ref[...], preferred_element_type=jnp.float32) \
        + bhh_ref[...]
    r = jax.nn.sigmoid(gi[:, :d] + gh[:, :d])
    z = jax.nn.sigmoid(gi[:, d:2 * d] + gh[:, d:2 * d])
    nn = jnp.tanh(gi[:, 2 * d:] + r * gh[:, 2 * d:])
    hn = (1.0 - z) * nn + z * h
    ho_ref[...] = hn
    if has_next:
        mo_ref[...] = jnp.dot(hn, wn_ref[...],
                              preferred_element_type=jnp.float32)


def _transform(h, w, bn):
    n, d = h.shape
    return pl.pallas_call(
        _mm_body,
        grid=(n // bn,),
        in_specs=[pl.BlockSpec((bn, d), lambda i: (i, 0)),
                  pl.BlockSpec((d, d), lambda i: (0, 0))],
        out_specs=pl.BlockSpec((bn, d), lambda i: (i, 0)),
        out_shape=jax.ShapeDtypeStruct((n, d), jnp.float32),
    )(h, w)


def _gru(agg2, h, wih_t, whh_t, bih, bhh, w_next, bn):
    n, d = h.shape
    nb = n // bn
    has_next = w_next is not None
    row = pl.BlockSpec((bn, d), lambda i: (i, 0))
    out_shapes = [jax.ShapeDtypeStruct((n, d), jnp.float32)]
    out_specs = [row]
    if has_next:
        out_shapes.append(jax.ShapeDtypeStruct((n, d), jnp.float32))
        out_specs.append(row)
    res = pl.pallas_call(
        functools.partial(_gru_body, has_next, d),
        grid=(nb,),
        in_specs=[
            pl.BlockSpec((bn, d), lambda i: (i, 0)),
            pl.BlockSpec((bn, d), lambda i: (i + nb, 0)),
            row,
            pl.BlockSpec((d, 3 * d), lambda i: (0, 0)),
            pl.BlockSpec((d, 3 * d), lambda i: (0, 0)),
            pl.BlockSpec((1, 3 * d), lambda i: (0, 0)),
            pl.BlockSpec((1, 3 * d), lambda i: (0, 0)),
            pl.BlockSpec((d, d), lambda i: (0, 0)),
        ],
        out_specs=out_specs,
        out_shape=out_shapes,
    )(agg2, agg2, h, wih_t, whh_t, bih, bhh,
      w_next if has_next else jnp.zeros((d, d), jnp.float32))
    return res if has_next else (res[0], None)


def kernel(x, edge_index, weight, W_ih, W_hh, b_ih, b_hh):
    n, d = x.shape
    e = edge_index.shape[1]
    num_layers = weight.shape[0]
    bn = 1000

    src = edge_index[0]
    dst = edge_index[1]
    wih_t = W_ih.T
    whh_t = W_hh.T
    bih = b_ih.reshape(1, -1)
    bhh = b_hh.reshape(1, -1)
    sc_scatter = _make_sc_scatter(n, e, d)

    h = x
    m = _transform(h, weight[0], bn)
    for i in range(num_layers):
        agg2 = sc_scatter(m, src, dst)
        w_next = weight[i + 1] if i + 1 < num_layers else None
        h, m = _gru(agg2, h, wih_t, whh_t, bih, bhh, w_next, bn)
    return h
